# baseline (speedup 1.0000x reference)
import functools

import jax
import jax.numpy as jnp
from jax import lax
from jax.experimental import pallas as pl
from jax.experimental.pallas import tpu as pltpu
from jax.experimental.pallas import tpu_sc as plsc

_NC = 2
_NS = 16
_NW = _NC * _NS

_LANES = 16
_BAGS_PER_GROUP = 2


def _sc_bag_sums_body(groups_per_w, bag, emb, table_hbm, text_hbm, out_hbm,
                      idx_v, rows0, rows1, rows2, rows3, out_v,
                      sem0, sem1, sem2, sem3):
    bags_per_w = groups_per_w * _BAGS_PER_GROUP
    cols = emb // _LANES

    wid = lax.axis_index("s") * _NC + lax.axis_index("c")

    pltpu.sync_copy(text_hbm.at[pl.ds(wid * groups_per_w, groups_per_w)], idx_v)

    bufs = (rows0, rows1, rows2, rows3)
    sems = (sem0, sem1, sem2, sem3)

    def gather(g, u):
        return pltpu.make_async_copy(table_hbm.at[idx_v.at[g]], bufs[u], sems[u])

    def reduce_group(g, buf):
        for b in range(_BAGS_PER_GROUP):
            r0 = b * bag
            accs = [buf[r0, pl.ds(c * _LANES, _LANES)] for c in range(cols)]
            for i in range(1, bag):
                for c in range(cols):
                    accs[c] = accs[c] + buf[r0 + i, pl.ds(c * _LANES, _LANES)]
            row = _BAGS_PER_GROUP * g + b
            for c in range(cols):
                out_v[row, pl.ds(c * _LANES, _LANES)] = accs[c]

    for u in range(3):
        gather(u, u).start()

    def body(j, carry):
        for u in range(4):
            g = 4 * j + u

            @pl.when(g + 3 < groups_per_w)
            def _():
                gather(g + 3, (u + 3) % 4).start()

            gather(g, u).wait()
            reduce_group(g, bufs[u])
        return carry

    lax.fori_loop(0, groups_per_w // 4, body, 0)

    pltpu.sync_copy(out_v, out_hbm.at[pl.ds(wid * bags_per_w, bags_per_w)])


def _sc_repack_body(V, emb, tabT_hbm, out_hbm, x0, x1, z0, z1, xt,
                    lsem0, lsem1, ssem0, ssem1):
    n_cols = V // 128
    tail = V - n_cols * 128

    wid = lax.axis_index("s") * _NC + lax.axis_index("c")

    iota = lax.iota(jnp.int32, 16)

    def load(c, buf, sem):
        return pltpu.make_async_copy(
            tabT_hbm.at[:, pl.ds(c * 128, 128)], buf, sem)

    def store(c, buf, sem):
        return pltpu.make_async_copy(
            buf, out_hbm.at[pl.ds(c * (128 // 2), 128 // 2)], sem)

    def repack(x, z, ntok):
        def block(b, carry):
            t0 = b * 16
            for e0 in range(0, emb, 16):
                ts, vals = [], []
                for s in range(16):
                    t = t0 + ((iota + s) & 15)
                    ts.append(t)
                    vals.append(plsc.load_gather(x, [e0 + iota, t]))
                for s in range(16):
                    t = ts[s]
                    wr_row = t >> 1
                    wr_col = ((t & 1) << 6) + (e0 + iota)
                    plsc.store_scatter(z, [wr_row, wr_col], vals[s])
            return carry

        lax.fori_loop(0, ntok // 16, block, 0)

    n_mine_max = (n_cols - 1) // _NW + 1

    def col(k):
        return wid + k * _NW

    @pl.when(col(0) < n_cols)
    def _():
        load(col(0), x0, lsem0).start()

    def body(j, carry):
        k0 = 2 * j
        k1 = k0 + 1

        @pl.when(col(k1) < n_cols)
        def _():
            load(col(k1), x1, lsem1).start()

        @pl.when(col(k0) < n_cols)
        def _():
            load(col(k0), x0, lsem0).wait()

            @pl.when(k0 >= 2)
            def _():
                store(0, z0, ssem0).wait()

            repack(x0, z0, 128)
            store(col(k0), z0, ssem0).start()

        @pl.when(col(k1 + 1) < n_cols)
        def _():
            load(col(k1 + 1), x0, lsem0).start()

        @pl.when(col(k1) < n_cols)
        def _():
            load(col(k1), x1, lsem1).wait()

            @pl.when(k1 >= 3)
            def _():
                store(0, z1, ssem1).wait()

            repack(x1, z1, 128)
            store(col(k1), z1, ssem1).start()

        return carry

    n_iter = (n_mine_max + 1) // 2
    lax.fori_loop(0, n_iter, body, 0)

    store(0, z0, ssem0).wait()
    store(0, z1, ssem1).wait()

    @pl.when((wid == 0) & (tail > 0))
    def _():
        pltpu.sync_copy(tabT_hbm.at[:, pl.ds(n_cols * 128, tail)], xt)
        repack(xt, z0, tail)
        pltpu.sync_copy(z0.at[pl.ds(0, tail // 2)],
                        out_hbm.at[pl.ds(n_cols * 64, tail // 2)])


def _mlp_body(inv_bag, x_ref, w1_ref, b1_ref, w2_ref, b2_ref, o_ref):
    x = x_ref[...] * inv_bag
    h = jnp.maximum(
        jnp.dot(x, w1_ref[...], preferred_element_type=jnp.float32)
        + b1_ref[...], 0.0)
    o_ref[...] = (
        jnp.dot(h, w2_ref[...], preferred_element_type=jnp.float32)
        + b2_ref[...])


def kernel(text, table, W1, b1, W2, b2):
    B, bag = text.shape
    V, emb = table.shape
    H = W1.shape[1]
    C = W2.shape[1]

    bags_per_w = B // _NW
    groups_per_w = bags_per_w // _BAGS_PER_GROUP
    idx_per_group = _BAGS_PER_GROUP * bag

    text_flat = jax.lax.optimization_barrier(
        text.astype(jnp.int32).reshape(B * bag))
    text2 = text_flat.reshape(_NW * groups_per_w, idx_per_group)

    mesh = plsc.VectorSubcoreMesh(core_axis_name="c", subcore_axis_name="s")

    tabT = table.T
    out128 = pl.kernel(
        functools.partial(_sc_repack_body, V, emb),
        out_type=jax.ShapeDtypeStruct((V // 2, 2 * emb), jnp.float32),
        mesh=mesh,
        compiler_params=pltpu.CompilerParams(
            use_tc_tiling_on_sc=True, needs_layout_passes=False),
        scratch_types=[
            pltpu.VMEM((emb, 128), jnp.float32),
            pltpu.VMEM((emb, 128), jnp.float32),
            pltpu.VMEM((64, 128), jnp.float32),
            pltpu.VMEM((64, 128), jnp.float32),
            pltpu.VMEM((64, 64), jnp.float32),
            pltpu.SemaphoreType.DMA,
            pltpu.SemaphoreType.DMA,
            pltpu.SemaphoreType.DMA,
            pltpu.SemaphoreType.DMA,
        ],
    )(tabT)
    table_lin = out128.reshape(V, emb)
    sums = pl.kernel(
        functools.partial(_sc_bag_sums_body, groups_per_w, bag, emb),
        out_type=jax.ShapeDtypeStruct((B, emb), jnp.float32),
        mesh=mesh,
        compiler_params=pltpu.CompilerParams(use_tc_tiling_on_sc=False),
        scratch_types=[
            pltpu.VMEM((groups_per_w, idx_per_group), jnp.int32),
            pltpu.VMEM((idx_per_group, emb), jnp.float32),
            pltpu.VMEM((idx_per_group, emb), jnp.float32),
            pltpu.VMEM((idx_per_group, emb), jnp.float32),
            pltpu.VMEM((idx_per_group, emb), jnp.float32),
            pltpu.VMEM((bags_per_w, emb), jnp.float32),
            pltpu.SemaphoreType.DMA,
            pltpu.SemaphoreType.DMA,
            pltpu.SemaphoreType.DMA,
            pltpu.SemaphoreType.DMA,
        ],
    )(table_lin, text2)

    BM = 1024
    logits = pl.pallas_call(
        functools.partial(_mlp_body, 1.0 / bag),
        grid=(B // BM,),
        in_specs=[
            pl.BlockSpec((BM, emb), lambda i: (i, 0)),
            pl.BlockSpec((emb, H), lambda i: (0, 0)),
            pl.BlockSpec((1, H), lambda i: (0, 0)),
            pl.BlockSpec((H, C), lambda i: (0, 0)),
            pl.BlockSpec((1, C), lambda i: (0, 0)),
        ],
        out_specs=pl.BlockSpec((BM, C), lambda i: (i, 0)),
        out_shape=jax.ShapeDtypeStruct((B, C), jnp.float32),
    )(sums, W1, b1.reshape(1, H), W2, b2.reshape(1, C))
    return logits


# speedup vs baseline: 1.0485x; 1.0485x over previous
import functools

import jax
import jax.numpy as jnp
from jax import lax
from jax.experimental import pallas as pl
from jax.experimental.pallas import tpu as pltpu
from jax.experimental.pallas import tpu_sc as plsc

_NC = 2
_NS = 16
_NW = _NC * _NS

_LANES = 16
_BAGS_PER_GROUP = 2


def _sc_bag_sums_body(groups_per_w, bag, emb, table_hbm, text_hbm, out_hbm,
                      idx_v, rows0, rows1, out_v, sem0, sem1):
    bags_per_w = groups_per_w * _BAGS_PER_GROUP
    cols = emb // _LANES

    wid = lax.axis_index("s") * _NC + lax.axis_index("c")

    pltpu.sync_copy(text_hbm.at[pl.ds(wid * groups_per_w, groups_per_w)], idx_v)

    def gather(g, buf, sem):
        return pltpu.make_async_copy(table_hbm.at[idx_v.at[g]], buf, sem)

    def reduce_group(g, buf):
        for b in range(_BAGS_PER_GROUP):
            r0 = b * bag
            accs = [buf[r0, pl.ds(c * _LANES, _LANES)] for c in range(cols)]
            for i in range(1, bag):
                for c in range(cols):
                    accs[c] = accs[c] + buf[r0 + i, pl.ds(c * _LANES, _LANES)]
            row = _BAGS_PER_GROUP * g + b
            for c in range(cols):
                out_v[row, pl.ds(c * _LANES, _LANES)] = accs[c]

    gather(0, rows0, sem0).start()

    def body(gg, carry):
        g0 = 2 * gg
        g1 = g0 + 1
        gather(g1, rows1, sem1).start()
        gather(g0, rows0, sem0).wait()
        reduce_group(g0, rows0)

        @pl.when(g1 + 1 < groups_per_w)
        def _():
            gather(g1 + 1, rows0, sem0).start()

        gather(g1, rows1, sem1).wait()
        reduce_group(g1, rows1)
        return carry

    lax.fori_loop(0, groups_per_w // 2, body, 0)

    pltpu.sync_copy(out_v, out_hbm.at[pl.ds(wid * bags_per_w, bags_per_w)])


def _sc_repack_body(V, emb, tabT_hbm, out_hbm, x0, x1, z0, z1, xt,
                    lsem0, lsem1, ssem0, ssem1):
    n_cols = V // 128
    tail = V - n_cols * 128

    wid = lax.axis_index("s") * _NC + lax.axis_index("c")

    iota = lax.iota(jnp.int32, 16)

    def load(c, buf, sem):
        return pltpu.make_async_copy(
            tabT_hbm.at[:, pl.ds(c * 128, 128)], buf, sem)

    def store(c, buf, sem):
        return pltpu.make_async_copy(
            buf, out_hbm.at[pl.ds(c * (128 // 2), 128 // 2)], sem)

    def repack(x, z, ntok):
        def block(b, carry):
            t0 = b * 16
            for e0 in range(0, emb, 16):
                ts, vals = [], []
                for s in range(16):
                    t = t0 + ((iota + s) & 15)
                    ts.append(t)
                    vals.append(plsc.load_gather(x, [e0 + iota, t]))
                for s in range(16):
                    t = ts[s]
                    wr_row = t >> 1
                    wr_col = ((t & 1) << 6) + (e0 + iota)
                    plsc.store_scatter(z, [wr_row, wr_col], vals[s])
            return carry

        lax.fori_loop(0, ntok // 16, block, 0)

    n_mine_max = (n_cols - 1) // _NW + 1

    def col(k):
        return wid + k * _NW

    @pl.when(col(0) < n_cols)
    def _():
        load(col(0), x0, lsem0).start()

    def body(j, carry):
        k0 = 2 * j
        k1 = k0 + 1

        @pl.when(col(k1) < n_cols)
        def _():
            load(col(k1), x1, lsem1).start()

        @pl.when(col(k0) < n_cols)
        def _():
            load(col(k0), x0, lsem0).wait()

            @pl.when(k0 >= 2)
            def _():
                store(0, z0, ssem0).wait()

            repack(x0, z0, 128)
            store(col(k0), z0, ssem0).start()

        @pl.when(col(k1 + 1) < n_cols)
        def _():
            load(col(k1 + 1), x0, lsem0).start()

        @pl.when(col(k1) < n_cols)
        def _():
            load(col(k1), x1, lsem1).wait()

            @pl.when(k1 >= 3)
            def _():
                store(0, z1, ssem1).wait()

            repack(x1, z1, 128)
            store(col(k1), z1, ssem1).start()

        return carry

    n_iter = (n_mine_max + 1) // 2
    lax.fori_loop(0, n_iter, body, 0)

    store(0, z0, ssem0).wait()
    store(0, z1, ssem1).wait()

    @pl.when((wid == 0) & (tail > 0))
    def _():
        pltpu.sync_copy(tabT_hbm.at[:, pl.ds(n_cols * 128, tail)], xt)
        repack(xt, z0, tail)
        pltpu.sync_copy(z0.at[pl.ds(0, tail // 2)],
                        out_hbm.at[pl.ds(n_cols * 64, tail // 2)])


def _mlp_body(inv_bag, x_ref, w1_ref, b1_ref, w2_ref, b2_ref, o_ref):
    x = x_ref[...] * inv_bag
    h = jnp.maximum(
        jnp.dot(x, w1_ref[...], preferred_element_type=jnp.float32)
        + b1_ref[...], 0.0)
    o_ref[...] = (
        jnp.dot(h, w2_ref[...], preferred_element_type=jnp.float32)
        + b2_ref[...])


def kernel(text, table, W1, b1, W2, b2):
    B, bag = text.shape
    V, emb = table.shape
    H = W1.shape[1]
    C = W2.shape[1]

    bags_per_w = B // _NW
    groups_per_w = bags_per_w // _BAGS_PER_GROUP
    idx_per_group = _BAGS_PER_GROUP * bag

    text_flat = jax.lax.optimization_barrier(
        text.astype(jnp.int32).reshape(B * bag))
    text2 = text_flat.reshape(_NW * groups_per_w, idx_per_group)

    mesh = plsc.VectorSubcoreMesh(core_axis_name="c", subcore_axis_name="s")

    tabT = table.T
    out128 = pl.kernel(
        functools.partial(_sc_repack_body, V, emb),
        out_type=jax.ShapeDtypeStruct((V // 2, 2 * emb), jnp.float32),
        mesh=mesh,
        compiler_params=pltpu.CompilerParams(
            use_tc_tiling_on_sc=True, needs_layout_passes=False),
        scratch_types=[
            pltpu.VMEM((emb, 128), jnp.float32),
            pltpu.VMEM((emb, 128), jnp.float32),
            pltpu.VMEM((64, 128), jnp.float32),
            pltpu.VMEM((64, 128), jnp.float32),
            pltpu.VMEM((64, 64), jnp.float32),
            pltpu.SemaphoreType.DMA,
            pltpu.SemaphoreType.DMA,
            pltpu.SemaphoreType.DMA,
            pltpu.SemaphoreType.DMA,
        ],
    )(tabT)
    table_lin = out128.reshape(V, emb)
    sums = pl.kernel(
        functools.partial(_sc_bag_sums_body, groups_per_w, bag, emb),
        out_type=jax.ShapeDtypeStruct((B, emb), jnp.float32),
        mesh=mesh,
        compiler_params=pltpu.CompilerParams(use_tc_tiling_on_sc=False),
        scratch_types=[
            pltpu.VMEM((groups_per_w, idx_per_group), jnp.int32),
            pltpu.VMEM((idx_per_group, emb), jnp.float32),
            pltpu.VMEM((idx_per_group, emb), jnp.float32),
            pltpu.VMEM((bags_per_w, emb), jnp.float32),
            pltpu.SemaphoreType.DMA,
            pltpu.SemaphoreType.DMA,
        ],
    )(table_lin, text2)

    BM = 1024
    logits = pl.pallas_call(
        functools.partial(_mlp_body, 1.0 / bag),
        grid=(B // BM,),
        in_specs=[
            pl.BlockSpec((BM, emb), lambda i: (i, 0)),
            pl.BlockSpec((emb, H), lambda i: (0, 0)),
            pl.BlockSpec((1, H), lambda i: (0, 0)),
            pl.BlockSpec((H, C), lambda i: (0, 0)),
            pl.BlockSpec((1, C), lambda i: (0, 0)),
        ],
        out_specs=pl.BlockSpec((BM, C), lambda i: (i, 0)),
        out_shape=jax.ShapeDtypeStruct((B, C), jnp.float32),
    )(sums, W1, b1.reshape(1, H), W2, b2.reshape(1, C))
    return logits


# speedup vs baseline: 1.1711x; 1.1169x over previous
import functools

import jax
import jax.numpy as jnp
from jax import lax
from jax.experimental import pallas as pl
from jax.experimental.pallas import tpu as pltpu
from jax.experimental.pallas import tpu_sc as plsc

_NC = 2
_NS = 16
_NW = _NC * _NS

_LANES = 16
_BAGS_PER_GROUP = 2


def _sc_bag_sums_body(groups_per_w, bag, emb, table_hbm, text_hbm, out_hbm,
                      idx_v, rows0, rows1, out_v, sem0, sem1):
    bags_per_w = groups_per_w * _BAGS_PER_GROUP
    cols = emb // _LANES

    wid = lax.axis_index("s") * _NC + lax.axis_index("c")

    pltpu.sync_copy(text_hbm.at[pl.ds(wid * groups_per_w, groups_per_w)], idx_v)

    def gather(g, buf, sem):
        return pltpu.make_async_copy(table_hbm.at[idx_v.at[g]], buf, sem)

    def reduce_group(g, buf):
        for b in range(_BAGS_PER_GROUP):
            r0 = b * bag
            accs = [buf[r0, pl.ds(c * _LANES, _LANES)] for c in range(cols)]
            for i in range(1, bag):
                for c in range(cols):
                    accs[c] = accs[c] + buf[r0 + i, pl.ds(c * _LANES, _LANES)]
            row = _BAGS_PER_GROUP * g + b
            for c in range(cols):
                out_v[row, pl.ds(c * _LANES, _LANES)] = accs[c]

    gather(0, rows0, sem0).start()

    def body(gg, carry):
        g0 = 2 * gg
        g1 = g0 + 1
        gather(g1, rows1, sem1).start()
        gather(g0, rows0, sem0).wait()
        reduce_group(g0, rows0)

        @pl.when(g1 + 1 < groups_per_w)
        def _():
            gather(g1 + 1, rows0, sem0).start()

        gather(g1, rows1, sem1).wait()
        reduce_group(g1, rows1)
        return carry

    lax.fori_loop(0, groups_per_w // 2, body, 0)

    pltpu.sync_copy(out_v, out_hbm.at[pl.ds(wid * bags_per_w, bags_per_w)])


def _sc_repack_body(V, emb, tabT_hbm, out_hbm, x0, x1, z0, z1, xt,
                    lsem0, lsem1, ssem0, ssem1):
    ntok_unit = 256
    n_cols = V // ntok_unit
    tail = V - n_cols * ntok_unit

    wid = lax.axis_index("s") * _NC + lax.axis_index("c")

    iota = lax.iota(jnp.int32, 16)

    def load(c, buf, sem):
        return pltpu.make_async_copy(
            tabT_hbm.at[:, pl.ds(c * ntok_unit, ntok_unit)], buf, sem)

    def store(c, buf, sem):
        return pltpu.make_async_copy(
            buf, out_hbm.at[pl.ds(c * (ntok_unit // 2), ntok_unit // 2)], sem)

    def repack(x, z, ntok):
        def block(b, carry):
            t0 = b * 16
            for e0 in range(0, emb, 16):
                ts, vals = [], []
                for s in range(16):
                    t = t0 + ((iota + s) & 15)
                    ts.append(t)
                    vals.append(plsc.load_gather(x, [e0 + iota, t]))
                for s in range(16):
                    t = ts[s]
                    wr_row = t >> 1
                    wr_col = ((t & 1) << 6) + (e0 + iota)
                    plsc.store_scatter(z, [wr_row, wr_col], vals[s])
            return carry

        lax.fori_loop(0, ntok // 16, block, 0)

    n_mine_max = (n_cols - 1) // _NW + 1

    def col(k):
        return wid + k * _NW

    @pl.when(col(0) < n_cols)
    def _():
        load(col(0), x0, lsem0).start()

    def body(j, carry):
        k0 = 2 * j
        k1 = k0 + 1

        @pl.when(col(k1) < n_cols)
        def _():
            load(col(k1), x1, lsem1).start()

        @pl.when(col(k0) < n_cols)
        def _():
            load(col(k0), x0, lsem0).wait()

            @pl.when(k0 >= 2)
            def _():
                store(0, z0, ssem0).wait()

            repack(x0, z0, ntok_unit)
            store(col(k0), z0, ssem0).start()

        @pl.when(col(k1 + 1) < n_cols)
        def _():
            load(col(k1 + 1), x0, lsem0).start()

        @pl.when(col(k1) < n_cols)
        def _():
            load(col(k1), x1, lsem1).wait()

            @pl.when(k1 >= 3)
            def _():
                store(0, z1, ssem1).wait()

            repack(x1, z1, ntok_unit)
            store(col(k1), z1, ssem1).start()

        return carry

    n_iter = (n_mine_max + 1) // 2
    lax.fori_loop(0, n_iter, body, 0)

    store(0, z0, ssem0).wait()
    store(0, z1, ssem1).wait()

    @pl.when((wid == 0) & (tail > 0))
    def _():
        pltpu.sync_copy(tabT_hbm.at[:, pl.ds(n_cols * ntok_unit, tail)], xt)
        repack(xt, z0, tail)
        pltpu.sync_copy(z0.at[pl.ds(0, tail // 2)],
                        out_hbm.at[pl.ds(n_cols * (ntok_unit // 2), tail // 2)])


def _mlp_body(inv_bag, x_ref, w1_ref, b1_ref, w2_ref, b2_ref, o_ref):
    x = x_ref[...] * inv_bag
    h = jnp.maximum(
        jnp.dot(x, w1_ref[...], preferred_element_type=jnp.float32)
        + b1_ref[...], 0.0)
    o_ref[...] = (
        jnp.dot(h, w2_ref[...], preferred_element_type=jnp.float32)
        + b2_ref[...])


def kernel(text, table, W1, b1, W2, b2):
    B, bag = text.shape
    V, emb = table.shape
    H = W1.shape[1]
    C = W2.shape[1]

    bags_per_w = B // _NW
    groups_per_w = bags_per_w // _BAGS_PER_GROUP
    idx_per_group = _BAGS_PER_GROUP * bag

    text_flat = jax.lax.optimization_barrier(
        text.astype(jnp.int32).reshape(B * bag))
    text2 = text_flat.reshape(_NW * groups_per_w, idx_per_group)

    mesh = plsc.VectorSubcoreMesh(core_axis_name="c", subcore_axis_name="s")

    tabT = table.T
    out128 = pl.kernel(
        functools.partial(_sc_repack_body, V, emb),
        out_type=jax.ShapeDtypeStruct((V // 2, 2 * emb), jnp.float32),
        mesh=mesh,
        compiler_params=pltpu.CompilerParams(
            use_tc_tiling_on_sc=True, needs_layout_passes=False),
        scratch_types=[
            pltpu.VMEM((emb, 256), jnp.float32),
            pltpu.VMEM((emb, 256), jnp.float32),
            pltpu.VMEM((128, 128), jnp.float32),
            pltpu.VMEM((128, 128), jnp.float32),
            pltpu.VMEM((64, 64), jnp.float32),
            pltpu.SemaphoreType.DMA,
            pltpu.SemaphoreType.DMA,
            pltpu.SemaphoreType.DMA,
            pltpu.SemaphoreType.DMA,
        ],
    )(tabT)
    table_lin = out128.reshape(V, emb)
    sums = pl.kernel(
        functools.partial(_sc_bag_sums_body, groups_per_w, bag, emb),
        out_type=jax.ShapeDtypeStruct((B, emb), jnp.float32),
        mesh=mesh,
        compiler_params=pltpu.CompilerParams(use_tc_tiling_on_sc=False),
        scratch_types=[
            pltpu.VMEM((groups_per_w, idx_per_group), jnp.int32),
            pltpu.VMEM((idx_per_group, emb), jnp.float32),
            pltpu.VMEM((idx_per_group, emb), jnp.float32),
            pltpu.VMEM((bags_per_w, emb), jnp.float32),
            pltpu.SemaphoreType.DMA,
            pltpu.SemaphoreType.DMA,
        ],
    )(table_lin, text2)

    BM = 1024
    logits = pl.pallas_call(
        functools.partial(_mlp_body, 1.0 / bag),
        grid=(B // BM,),
        in_specs=[
            pl.BlockSpec((BM, emb), lambda i: (i, 0)),
            pl.BlockSpec((emb, H), lambda i: (0, 0)),
            pl.BlockSpec((1, H), lambda i: (0, 0)),
            pl.BlockSpec((H, C), lambda i: (0, 0)),
            pl.BlockSpec((1, C), lambda i: (0, 0)),
        ],
        out_specs=pl.BlockSpec((BM, C), lambda i: (i, 0)),
        out_shape=jax.ShapeDtypeStruct((B, C), jnp.float32),
    )(sums, W1, b1.reshape(1, H), W2, b2.reshape(1, C))
    return logits


# speedup vs baseline: 1.1774x; 1.0054x over previous
import functools

import jax
import jax.numpy as jnp
from jax import lax
from jax.experimental import pallas as pl
from jax.experimental.pallas import tpu as pltpu
from jax.experimental.pallas import tpu_sc as plsc

_NC = 2
_NS = 16
_NW = _NC * _NS

_LANES = 16
_BAGS_PER_GROUP = 2


def _sc_bag_sums_body(groups_per_w, bag, emb, table_hbm, text_hbm, out_hbm,
                      idx_v, rows0, rows1, out_v, sem0, sem1):
    bags_per_w = groups_per_w * _BAGS_PER_GROUP
    cols = emb // _LANES

    wid = lax.axis_index("s") * _NC + lax.axis_index("c")

    pltpu.sync_copy(text_hbm.at[pl.ds(wid * groups_per_w, groups_per_w)], idx_v)

    def gather(g, buf, sem):
        return pltpu.make_async_copy(table_hbm.at[idx_v.at[g]], buf, sem)

    def reduce_group(g, buf):
        for b in range(_BAGS_PER_GROUP):
            r0 = b * bag
            accs = [buf[r0, pl.ds(c * _LANES, _LANES)] for c in range(cols)]
            for i in range(1, bag):
                for c in range(cols):
                    accs[c] = accs[c] + buf[r0 + i, pl.ds(c * _LANES, _LANES)]
            row = _BAGS_PER_GROUP * g + b
            for c in range(cols):
                out_v[row, pl.ds(c * _LANES, _LANES)] = accs[c]

    gather(0, rows0, sem0).start()

    def body(gg, carry):
        g0 = 2 * gg
        g1 = g0 + 1
        gather(g1, rows1, sem1).start()
        gather(g0, rows0, sem0).wait()
        reduce_group(g0, rows0)

        @pl.when(g1 + 1 < groups_per_w)
        def _():
            gather(g1 + 1, rows0, sem0).start()

        gather(g1, rows1, sem1).wait()
        reduce_group(g1, rows1)
        return carry

    lax.fori_loop(0, groups_per_w // 2, body, 0)

    pltpu.sync_copy(out_v, out_hbm.at[pl.ds(wid * bags_per_w, bags_per_w)])


def _sc_repack_body(V, emb, tabT_hbm, out_hbm, x0, x1, z0, z1, xt,
                    lsem0, lsem1, ssem0, ssem1):
    ntok_unit = 384
    n_cols = V // ntok_unit
    tail = V - n_cols * ntok_unit

    wid = lax.axis_index("s") * _NC + lax.axis_index("c")

    iota = lax.iota(jnp.int32, 16)

    def load(c, buf, sem):
        return pltpu.make_async_copy(
            tabT_hbm.at[:, pl.ds(c * ntok_unit, ntok_unit)], buf, sem)

    def store(c, buf, sem):
        return pltpu.make_async_copy(
            buf, out_hbm.at[pl.ds(c * (ntok_unit // 2), ntok_unit // 2)], sem)

    def repack(x, z, ntok):
        def block(b, carry):
            t0 = b * 16
            for e0 in range(0, emb, 16):
                ts, vals = [], []
                for s in range(16):
                    t = t0 + ((iota + s) & 15)
                    ts.append(t)
                    vals.append(plsc.load_gather(x, [e0 + iota, t]))
                for s in range(16):
                    t = ts[s]
                    wr_row = t >> 1
                    wr_col = ((t & 1) << 6) + (e0 + iota)
                    plsc.store_scatter(z, [wr_row, wr_col], vals[s])
            return carry

        lax.fori_loop(0, ntok // 16, block, 0)

    n_mine_max = (n_cols - 1) // _NW + 1

    def col(k):
        return wid + k * _NW

    @pl.when(col(0) < n_cols)
    def _():
        load(col(0), x0, lsem0).start()

    def body(j, carry):
        k0 = 2 * j
        k1 = k0 + 1

        @pl.when(col(k1) < n_cols)
        def _():
            load(col(k1), x1, lsem1).start()

        @pl.when(col(k0) < n_cols)
        def _():
            load(col(k0), x0, lsem0).wait()

            @pl.when(k0 >= 2)
            def _():
                store(0, z0, ssem0).wait()

            repack(x0, z0, ntok_unit)
            store(col(k0), z0, ssem0).start()

        @pl.when(col(k1 + 1) < n_cols)
        def _():
            load(col(k1 + 1), x0, lsem0).start()

        @pl.when(col(k1) < n_cols)
        def _():
            load(col(k1), x1, lsem1).wait()

            @pl.when(k1 >= 3)
            def _():
                store(0, z1, ssem1).wait()

            repack(x1, z1, ntok_unit)
            store(col(k1), z1, ssem1).start()

        return carry

    n_iter = (n_mine_max + 1) // 2
    lax.fori_loop(0, n_iter, body, 0)

    store(0, z0, ssem0).wait()
    store(0, z1, ssem1).wait()

    @pl.when((wid == 0) & (tail > 0))
    def _():
        pltpu.sync_copy(tabT_hbm.at[:, pl.ds(n_cols * ntok_unit, tail)], xt)
        repack(xt, z0, tail)
        pltpu.sync_copy(z0.at[pl.ds(0, tail // 2)],
                        out_hbm.at[pl.ds(n_cols * (ntok_unit // 2), tail // 2)])


def _mlp_body(inv_bag, x_ref, w1_ref, b1_ref, w2_ref, b2_ref, o_ref):
    x = x_ref[...] * inv_bag
    h = jnp.maximum(
        jnp.dot(x, w1_ref[...], preferred_element_type=jnp.float32)
        + b1_ref[...], 0.0)
    o_ref[...] = (
        jnp.dot(h, w2_ref[...], preferred_element_type=jnp.float32)
        + b2_ref[...])


def kernel(text, table, W1, b1, W2, b2):
    B, bag = text.shape
    V, emb = table.shape
    H = W1.shape[1]
    C = W2.shape[1]

    bags_per_w = B // _NW
    groups_per_w = bags_per_w // _BAGS_PER_GROUP
    idx_per_group = _BAGS_PER_GROUP * bag

    text_flat = jax.lax.optimization_barrier(
        text.astype(jnp.int32).reshape(B * bag))
    text2 = text_flat.reshape(_NW * groups_per_w, idx_per_group)

    mesh = plsc.VectorSubcoreMesh(core_axis_name="c", subcore_axis_name="s")

    tabT = table.T
    out128 = pl.kernel(
        functools.partial(_sc_repack_body, V, emb),
        out_type=jax.ShapeDtypeStruct((V // 2, 2 * emb), jnp.float32),
        mesh=mesh,
        compiler_params=pltpu.CompilerParams(
            use_tc_tiling_on_sc=True, needs_layout_passes=False),
        scratch_types=[
            pltpu.VMEM((emb, 384), jnp.float32),
            pltpu.VMEM((emb, 384), jnp.float32),
            pltpu.VMEM((192, 128), jnp.float32),
            pltpu.VMEM((192, 128), jnp.float32),
            pltpu.VMEM((64, 64), jnp.float32),
            pltpu.SemaphoreType.DMA,
            pltpu.SemaphoreType.DMA,
            pltpu.SemaphoreType.DMA,
            pltpu.SemaphoreType.DMA,
        ],
    )(tabT)
    table_lin = out128.reshape(V, emb)
    sums = pl.kernel(
        functools.partial(_sc_bag_sums_body, groups_per_w, bag, emb),
        out_type=jax.ShapeDtypeStruct((B, emb), jnp.float32),
        mesh=mesh,
        compiler_params=pltpu.CompilerParams(use_tc_tiling_on_sc=False),
        scratch_types=[
            pltpu.VMEM((groups_per_w, idx_per_group), jnp.int32),
            pltpu.VMEM((idx_per_group, emb), jnp.float32),
            pltpu.VMEM((idx_per_group, emb), jnp.float32),
            pltpu.VMEM((bags_per_w, emb), jnp.float32),
            pltpu.SemaphoreType.DMA,
            pltpu.SemaphoreType.DMA,
        ],
    )(table_lin, text2)

    BM = 1024
    logits = pl.pallas_call(
        functools.partial(_mlp_body, 1.0 / bag),
        grid=(B // BM,),
        in_specs=[
            pl.BlockSpec((BM, emb), lambda i: (i, 0)),
            pl.BlockSpec((emb, H), lambda i: (0, 0)),
            pl.BlockSpec((1, H), lambda i: (0, 0)),
            pl.BlockSpec((H, C), lambda i: (0, 0)),
            pl.BlockSpec((1, C), lambda i: (0, 0)),
        ],
        out_specs=pl.BlockSpec((BM, C), lambda i: (i, 0)),
        out_shape=jax.ShapeDtypeStruct((B, C), jnp.float32),
    )(sums, W1, b1.reshape(1, H), W2, b2.reshape(1, C))
    return logits
